# baseline (speedup 1.0000x reference)
import functools

import jax
import jax.numpy as jnp
from jax import lax
from jax.experimental import pallas as pl
from jax.experimental.pallas import tpu as pltpu
from jax.experimental.pallas import tpu_sc as plsc

NC = 2
NS = 16
NW = NC * NS

CHUNK = 16
RING = 4


@functools.partial(jax.jit, static_argnames=("n_tok", "d_model"))
def _embed_lookup(tok_ids, pos_ids, wte, wpe, *, n_tok, d_model):
    per_w = n_tok // NW
    n_chunks = per_w // CHUNK
    assert n_chunks % RING == 0 and n_chunks >= RING
    d_regs = d_model // 16

    mesh = plsc.VectorSubcoreMesh(
        core_axis_name="c", subcore_axis_name="s", num_cores=NC, num_subcores=NS
    )

    @functools.partial(
        pl.kernel,
        out_type=jax.ShapeDtypeStruct((n_tok, d_model), jnp.float32),
        mesh=mesh,
        scratch_types=[
            pltpu.VMEM((n_chunks, CHUNK), jnp.int32),
            pltpu.VMEM((n_chunks, CHUNK), jnp.int32),
            pltpu.VMEM((RING, CHUNK, d_model), jnp.float32),
            pltpu.VMEM((2, CHUNK, d_model), jnp.float32),
            pltpu.SemaphoreType.DMA((RING,)),
            pltpu.SemaphoreType.DMA((2,)),
            pltpu.SemaphoreType.DMA((RING,)),
        ],
    )
    def k(tok_hbm, pos_hbm, wte_hbm, wpe_hbm, out_hbm,
          tok_v, pos_v, rows_t, rows_p, sem_t, sem_p, sem_o):
        cid = lax.axis_index("c")
        sid = lax.axis_index("s")
        wid = sid * NC + cid
        base = wid * per_w
        pltpu.sync_copy(tok_hbm.at[wid], tok_v)
        pltpu.sync_copy(pos_hbm.at[wid], pos_v)

        for jj in range(2):
            pltpu.async_copy(wte_hbm.at[tok_v.at[jj]], rows_t.at[jj], sem_t.at[jj])
            pltpu.async_copy(wpe_hbm.at[pos_v.at[jj]], rows_p.at[jj], sem_p.at[jj])

        @pl.loop(0, n_chunks, step=RING)
        def _chunks(j):
            for b in range(RING):
                jj = j + b
                pb = b % 2
                tb2 = (b + 2) % RING
                pltpu.make_async_copy(
                    wte_hbm.at[tok_v.at[jj]], rows_t.at[b], sem_t.at[b]).wait()
                pltpu.make_async_copy(
                    wpe_hbm.at[pos_v.at[jj]], rows_p.at[pb], sem_p.at[pb]).wait()

                @pl.when(jj >= 2)
                def _():
                    for _h in range(2):
                        pltpu.make_async_copy(
                            rows_t.at[tb2, pl.ds(0, CHUNK // 2)],
                            out_hbm.at[pl.ds(base, CHUNK // 2)],
                            sem_o.at[tb2]).wait()

                @pl.when(jj + 2 < n_chunks)
                def _():
                    pltpu.async_copy(
                        wte_hbm.at[tok_v.at[jj + 2]], rows_t.at[tb2], sem_t.at[tb2])

                half = CHUNK // 2
                for h in range(2):
                    @plsc.parallel_loop(h * half, (h + 1) * half)
                    def _add_row(t):
                        for d in range(d_regs):
                            sl = pl.ds(d * 16, 16)
                            rows_t[b, t, sl] = rows_t[b, t, sl] + rows_p[pb, t, sl]

                    pltpu.async_copy(
                        rows_t.at[b, pl.ds(h * half, half)],
                        out_hbm.at[pl.ds(base + jj * CHUNK + h * half, half)],
                        sem_o.at[b])

                @pl.when(jj + 2 < n_chunks)
                def _():
                    pltpu.async_copy(
                        wpe_hbm.at[pos_v.at[jj + 2]], rows_p.at[pb], sem_p.at[pb])

        for b in (2, 3):
            for _h in range(2):
                pltpu.make_async_copy(
                    rows_t.at[b, pl.ds(0, CHUNK // 2)],
                    out_hbm.at[pl.ds(base, CHUNK // 2)],
                    sem_o.at[b]).wait()

    tok3 = tok_ids.reshape(NW, n_chunks, CHUNK)
    pos3 = pos_ids.reshape(NW, n_chunks, CHUNK)
    return k(tok3, pos3, wte, wpe)


def kernel(input_ids, position_ids, wte, wpe):
    b, s = input_ids.shape
    d = wte.shape[1]
    out = _embed_lookup(
        input_ids.reshape(-1).astype(jnp.int32),
        position_ids.reshape(-1).astype(jnp.int32),
        wte,
        wpe,
        n_tok=b * s,
        d_model=d,
    )
    return out.reshape(b, s, d)


# speedup vs baseline: 1.2859x; 1.2859x over previous
import functools

import jax
import jax.numpy as jnp
from jax import lax
from jax.experimental import pallas as pl
from jax.experimental.pallas import tpu as pltpu
from jax.experimental.pallas import tpu_sc as plsc

NC = 2
NS = 16
NW = NC * NS

CHUNK = 16
RING = 4


@functools.partial(jax.jit, static_argnames=("n_tok", "d_model"))
def _embed_lookup(tok_ids, pos_ids, wte, wpe, *, n_tok, d_model):
    per_w = n_tok // NW
    n_chunks = per_w // CHUNK
    assert n_chunks % RING == 0 and n_chunks >= RING
    d_regs = d_model // 16

    mesh = plsc.VectorSubcoreMesh(
        core_axis_name="c", subcore_axis_name="s", num_cores=NC, num_subcores=NS
    )

    @functools.partial(
        pl.kernel,
        out_type=jax.ShapeDtypeStruct((n_tok, d_model), jnp.float32),
        mesh=mesh,
        scratch_types=[
            pltpu.VMEM((n_chunks, CHUNK), jnp.int32),
            pltpu.VMEM((n_chunks, CHUNK), jnp.int32),
            pltpu.VMEM((RING, CHUNK, d_model), jnp.float32),
            pltpu.VMEM((2, CHUNK, d_model), jnp.float32),
            pltpu.SemaphoreType.DMA((RING,)),
            pltpu.SemaphoreType.DMA((2,)),
            pltpu.SemaphoreType.DMA((RING,)),
        ],
    )
    def k(tok_hbm, pos_hbm, wte_hbm, wpe_hbm, out_hbm,
          tok_v, pos_v, rows_t, rows_p, sem_t, sem_p, sem_o):
        cid = lax.axis_index("c")
        sid = lax.axis_index("s")
        wid = sid * NC + cid
        base = wid * per_w
        pltpu.sync_copy(tok_hbm.at[wid], tok_v)
        pltpu.sync_copy(pos_hbm.at[wid], pos_v)

        for jj in range(2):
            pltpu.async_copy(wte_hbm.at[tok_v.at[jj]], rows_t.at[jj], sem_t.at[jj])
        pltpu.async_copy(wpe_hbm.at[pos_v.at[0]], rows_p.at[0], sem_p.at[0])

        @pl.loop(0, n_chunks, step=RING)
        def _chunks(j):
            for b in range(RING):
                jj = j + b
                pb = b % 2
                tb2 = (b + 2) % RING
                pltpu.make_async_copy(
                    wte_hbm.at[tok_v.at[jj]], rows_t.at[b], sem_t.at[b]).wait()
                pltpu.make_async_copy(
                    wpe_hbm.at[pos_v.at[jj]], rows_p.at[pb], sem_p.at[pb]).wait()

                @pl.when(jj >= 2)
                def _():
                    pltpu.make_async_copy(
                        rows_t.at[tb2],
                        out_hbm.at[pl.ds(base, CHUNK)],
                        sem_o.at[tb2]).wait()

                @pl.when(jj + 2 < n_chunks)
                def _():
                    pltpu.async_copy(
                        wte_hbm.at[tok_v.at[jj + 2]], rows_t.at[tb2], sem_t.at[tb2])

                @pl.when(jj + 1 < n_chunks)
                def _():
                    pltpu.async_copy(
                        wpe_hbm.at[pos_v.at[jj + 1]], rows_p.at[1 - pb],
                        sem_p.at[1 - pb])

                @plsc.parallel_loop(0, CHUNK)
                def _add_row(t):
                    for d in range(d_regs):
                        sl = pl.ds(d * 16, 16)
                        rows_t[b, t, sl] = rows_t[b, t, sl] + rows_p[pb, t, sl]

                pltpu.async_copy(
                    rows_t.at[b],
                    out_hbm.at[pl.ds(base + jj * CHUNK, CHUNK)],
                    sem_o.at[b])


        for b in (2, 3):
            pltpu.make_async_copy(
                rows_t.at[b], out_hbm.at[pl.ds(base, CHUNK)], sem_o.at[b]).wait()

    tok3 = tok_ids.reshape(NW, n_chunks, CHUNK)
    pos3 = pos_ids.reshape(NW, n_chunks, CHUNK)
    return k(tok3, pos3, wte, wpe)


def kernel(input_ids, position_ids, wte, wpe):
    b, s = input_ids.shape
    d = wte.shape[1]
    out = _embed_lookup(
        input_ids.reshape(-1).astype(jnp.int32),
        position_ids.reshape(-1).astype(jnp.int32),
        wte,
        wpe,
        n_tok=b * s,
        d_model=d,
    )
    return out.reshape(b, s, d)


# speedup vs baseline: 1.3086x; 1.0176x over previous
import functools

import jax
import jax.numpy as jnp
from jax import lax
from jax.experimental import pallas as pl
from jax.experimental.pallas import tpu as pltpu
from jax.experimental.pallas import tpu_sc as plsc

NC = 2
NS = 16
NW = NC * NS

CHUNK = 16
RING = 4


@functools.partial(jax.jit, static_argnames=("n_tok", "d_model"))
def _embed_lookup(tok_ids, pos_ids, wte, wpe, *, n_tok, d_model):
    per_w = n_tok // NW
    n_chunks = per_w // CHUNK
    assert n_chunks % RING == 0 and n_chunks >= RING
    d_regs = d_model // 16

    mesh = plsc.VectorSubcoreMesh(
        core_axis_name="c", subcore_axis_name="s", num_cores=NC, num_subcores=NS
    )

    @functools.partial(
        pl.kernel,
        out_type=jax.ShapeDtypeStruct((n_tok, d_model), jnp.float32),
        mesh=mesh,
        scratch_types=[
            pltpu.VMEM((n_chunks, CHUNK), jnp.int32),
            pltpu.VMEM((n_chunks, CHUNK), jnp.int32),
            pltpu.VMEM((RING, CHUNK, d_model), jnp.float32),
            pltpu.VMEM((2, CHUNK, d_model), jnp.float32),
            pltpu.SemaphoreType.DMA((RING,)),
            pltpu.SemaphoreType.DMA((2,)),
            pltpu.SemaphoreType.DMA((RING,)),
        ],
    )
    def k(tok_hbm, pos_hbm, wte_hbm, wpe_hbm, out_hbm,
          tok_v, pos_v, rows_t, rows_p, sem_t, sem_p, sem_o):
        cid = lax.axis_index("c")
        sid = lax.axis_index("s")
        wid = sid * NC + cid
        base = wid * per_w
        cp_tok = pltpu.async_copy(tok_hbm.at[wid], tok_v, sem_t.at[0])
        cp_pos = pltpu.async_copy(pos_hbm.at[wid], pos_v, sem_p.at[0])
        cp_tok.wait()
        cp_pos.wait()

        for jj in range(2):
            pltpu.async_copy(wte_hbm.at[tok_v.at[jj]], rows_t.at[jj], sem_t.at[jj])
            pltpu.async_copy(wpe_hbm.at[pos_v.at[jj]], rows_p.at[jj], sem_p.at[jj])

        @pl.loop(0, n_chunks, step=RING)
        def _chunks(j):
            for b in range(RING):
                jj = j + b
                pb = b % 2
                tb2 = (b + 2) % RING
                pltpu.make_async_copy(
                    wte_hbm.at[tok_v.at[jj]], rows_t.at[b], sem_t.at[b]).wait()
                pltpu.make_async_copy(
                    wpe_hbm.at[pos_v.at[jj]], rows_p.at[pb], sem_p.at[pb]).wait()

                @pl.when(jj >= 2)
                def _():
                    pltpu.make_async_copy(
                        rows_t.at[tb2],
                        out_hbm.at[pl.ds(base, CHUNK)],
                        sem_o.at[tb2]).wait()

                @pl.when(jj + 2 < n_chunks)
                def _():
                    pltpu.async_copy(
                        wte_hbm.at[tok_v.at[jj + 2]], rows_t.at[tb2], sem_t.at[tb2])

                @plsc.parallel_loop(0, CHUNK)
                def _add_row(t):
                    for d in range(d_regs):
                        sl = pl.ds(d * 16, 16)
                        rows_t[b, t, sl] = rows_t[b, t, sl] + rows_p[pb, t, sl]

                pltpu.async_copy(
                    rows_t.at[b],
                    out_hbm.at[pl.ds(base + jj * CHUNK, CHUNK)],
                    sem_o.at[b])

                @pl.when(jj + 2 < n_chunks)
                def _():
                    pltpu.async_copy(
                        wpe_hbm.at[pos_v.at[jj + 2]], rows_p.at[pb], sem_p.at[pb])

        for b in (2, 3):
            pltpu.make_async_copy(
                rows_t.at[b], out_hbm.at[pl.ds(base, CHUNK)], sem_o.at[b]).wait()

    tok3 = tok_ids.reshape(NW, n_chunks, CHUNK)
    pos3 = pos_ids.reshape(NW, n_chunks, CHUNK)
    return k(tok3, pos3, wte, wpe)


def kernel(input_ids, position_ids, wte, wpe):
    b, s = input_ids.shape
    d = wte.shape[1]
    out = _embed_lookup(
        input_ids.reshape(-1).astype(jnp.int32),
        position_ids.reshape(-1).astype(jnp.int32),
        wte,
        wpe,
        n_tok=b * s,
        d_model=d,
    )
    return out.reshape(b, s, d)


# speedup vs baseline: 1.3183x; 1.0075x over previous
import functools

import jax
import jax.numpy as jnp
from jax import lax
from jax.experimental import pallas as pl
from jax.experimental.pallas import tpu as pltpu
from jax.experimental.pallas import tpu_sc as plsc

NC = 2
NS = 16
NW = NC * NS

SUP = 32
HALF = 16


@functools.partial(jax.jit, static_argnames=("n_tok", "d_model"))
def _embed_lookup(tok_ids, pos_ids, wte, wpe, *, n_tok, d_model):
    per_w = n_tok // NW
    n_sup = per_w // SUP
    assert n_sup % 2 == 0 and n_sup >= 2
    d_regs = d_model // 16

    mesh = plsc.VectorSubcoreMesh(
        core_axis_name="c", subcore_axis_name="s", num_cores=NC, num_subcores=NS
    )

    @functools.partial(
        pl.kernel,
        out_type=jax.ShapeDtypeStruct((n_tok, d_model), jnp.float32),
        mesh=mesh,
        scratch_types=[
            pltpu.VMEM((n_sup, SUP), jnp.int32),
            pltpu.VMEM((2 * n_sup, HALF), jnp.int32),
            pltpu.VMEM((2, SUP, d_model), jnp.float32),
            pltpu.VMEM((2, HALF, d_model), jnp.float32),
            pltpu.SemaphoreType.DMA((2,)),
            pltpu.SemaphoreType.DMA((2,)),
            pltpu.SemaphoreType.DMA((2,)),
        ],
    )
    def k(tok_hbm, pos_hbm, wte_hbm, wpe_hbm, out_hbm,
          tok_v, pos_v, rows_t, rows_p, sem_t, sem_p, sem_o):
        cid = lax.axis_index("c")
        sid = lax.axis_index("s")
        wid = sid * NC + cid
        base = wid * per_w
        cp_tok = pltpu.async_copy(tok_hbm.at[wid], tok_v, sem_t.at[0])
        cp_pos = pltpu.async_copy(pos_hbm.at[wid], pos_v, sem_p.at[0])
        cp_tok.wait()
        cp_pos.wait()

        pltpu.async_copy(wte_hbm.at[tok_v.at[0]], rows_t.at[0], sem_t.at[0])
        for h in range(2):
            pltpu.async_copy(wpe_hbm.at[pos_v.at[h]], rows_p.at[h], sem_p.at[h])

        @pl.loop(0, n_sup, step=2)
        def _supers(j):
            for sg in range(2):
                g = j + sg
                so2 = 1 - sg
                pltpu.make_async_copy(
                    wte_hbm.at[tok_v.at[g]], rows_t.at[sg], sem_t.at[sg]).wait()

                @pl.when(g >= 1)
                def _():
                    pltpu.make_async_copy(
                        rows_t.at[so2],
                        out_hbm.at[pl.ds(base, SUP)],
                        sem_o.at[so2]).wait()

                @pl.when(g + 1 < n_sup)
                def _():
                    pltpu.async_copy(
                        wte_hbm.at[tok_v.at[g + 1]], rows_t.at[so2],
                        sem_t.at[so2])

                for h in range(2):
                    hh = 2 * g + h
                    pltpu.make_async_copy(
                        wpe_hbm.at[pos_v.at[hh]], rows_p.at[h],
                        sem_p.at[h]).wait()

                    @plsc.parallel_loop(0, HALF)
                    def _add_row(t):
                        t2 = t + h * HALF
                        for d in range(d_regs):
                            sl = pl.ds(d * 16, 16)
                            rows_t[sg, t2, sl] = (
                                rows_t[sg, t2, sl] + rows_p[h, t, sl])

                    @pl.when(hh + 2 < 2 * n_sup)
                    def _():
                        pltpu.async_copy(
                            wpe_hbm.at[pos_v.at[hh + 2]], rows_p.at[h],
                            sem_p.at[h])

                pltpu.async_copy(
                    rows_t.at[sg],
                    out_hbm.at[pl.ds(base + g * SUP, SUP)],
                    sem_o.at[sg])

        pltpu.make_async_copy(
            rows_t.at[1], out_hbm.at[pl.ds(base, SUP)], sem_o.at[1]).wait()

    tok3 = tok_ids.reshape(NW, n_sup, SUP)
    pos3 = pos_ids.reshape(NW, 2 * n_sup, HALF)
    return k(tok3, pos3, wte, wpe)


def kernel(input_ids, position_ids, wte, wpe):
    b, s = input_ids.shape
    d = wte.shape[1]
    out = _embed_lookup(
        input_ids.reshape(-1).astype(jnp.int32),
        position_ids.reshape(-1).astype(jnp.int32),
        wte,
        wpe,
        n_tok=b * s,
        d_model=d,
    )
    return out.reshape(b, s, d)


# speedup vs baseline: 1.3737x; 1.0420x over previous
import functools

import jax
import jax.numpy as jnp
from jax import lax
from jax.experimental import pallas as pl
from jax.experimental.pallas import tpu as pltpu
from jax.experimental.pallas import tpu_sc as plsc

NC = 2
NS = 16
NW = NC * NS

SUP = 32
HALF = 16


@functools.partial(jax.jit, static_argnames=("n_tok", "d_model"))
def _embed_lookup(tok_ids, pos_ids, wte, wpe, *, n_tok, d_model):
    per_w = n_tok // NW
    n_sup = per_w // SUP
    assert n_sup % 2 == 0 and n_sup >= 2
    d_regs = d_model // 16

    mesh = plsc.VectorSubcoreMesh(
        core_axis_name="c", subcore_axis_name="s", num_cores=NC, num_subcores=NS
    )

    @functools.partial(
        pl.kernel,
        out_type=jax.ShapeDtypeStruct((n_tok, d_model), jnp.float32),
        mesh=mesh,
        scratch_types=[
            pltpu.VMEM((n_sup, SUP), jnp.int32),
            pltpu.VMEM((2 * n_sup, HALF), jnp.int32),
            pltpu.VMEM((2, SUP, d_model), jnp.float32),
            pltpu.VMEM((2, HALF, d_model), jnp.float32),
            pltpu.SemaphoreType.DMA((2,)),
            pltpu.SemaphoreType.DMA((2,)),
            pltpu.SemaphoreType.DMA((2,)),
        ],
    )
    def k(tok_hbm, pos_hbm, wte_hbm, wpe_hbm, out_hbm,
          tok_v, pos_v, rows_t, rows_p, sem_t, sem_p, sem_o):
        cid = lax.axis_index("c")
        sid = lax.axis_index("s")
        wid = sid * NC + cid
        base = wid * per_w
        cp_tok = pltpu.async_copy(tok_hbm.at[wid], tok_v, sem_t.at[0])
        cp_pos = pltpu.async_copy(pos_hbm.at[wid], pos_v, sem_p.at[0])
        cp_tok.wait()
        cp_pos.wait()

        pltpu.async_copy(wte_hbm.at[tok_v.at[0]], rows_t.at[0], sem_t.at[0])
        for h in range(2):
            pltpu.async_copy(wpe_hbm.at[pos_v.at[h]], rows_p.at[h], sem_p.at[h])

        @pl.loop(0, n_sup, step=2)
        def _supers(j):
            for sg in range(2):
                g = j + sg
                so2 = 1 - sg
                pltpu.make_async_copy(
                    wte_hbm.at[tok_v.at[g]], rows_t.at[sg], sem_t.at[sg]).wait()

                @pl.when(g >= 1)
                def _():
                    pltpu.make_async_copy(
                        rows_t.at[so2],
                        out_hbm.at[pl.ds(base, SUP)],
                        sem_o.at[so2]).wait()

                @pl.when(g + 1 < n_sup)
                def _():
                    pltpu.async_copy(
                        wte_hbm.at[tok_v.at[g + 1]], rows_t.at[so2],
                        sem_t.at[so2])

                for h in range(2):
                    hh = 2 * g + h
                    pltpu.make_async_copy(
                        wpe_hbm.at[pos_v.at[hh]], rows_p.at[h],
                        sem_p.at[h]).wait()

                    @plsc.parallel_loop(0, d_regs)
                    def _add_col(d):
                        sl = pl.ds(d * 16, 16)
                        for t in range(HALF):
                            t2 = t + h * HALF
                            rows_t[sg, t2, sl] = (
                                rows_t[sg, t2, sl] + rows_p[h, t, sl])

                    @pl.when(hh + 2 < 2 * n_sup)
                    def _():
                        pltpu.async_copy(
                            wpe_hbm.at[pos_v.at[hh + 2]], rows_p.at[h],
                            sem_p.at[h])

                pltpu.async_copy(
                    rows_t.at[sg],
                    out_hbm.at[pl.ds(base + g * SUP, SUP)],
                    sem_o.at[sg])

        pltpu.make_async_copy(
            rows_t.at[1], out_hbm.at[pl.ds(base, SUP)], sem_o.at[1]).wait()

    tok3 = tok_ids.reshape(NW, n_sup, SUP)
    pos3 = pos_ids.reshape(NW, 2 * n_sup, HALF)
    return k(tok3, pos3, wte, wpe)


def kernel(input_ids, position_ids, wte, wpe):
    b, s = input_ids.shape
    d = wte.shape[1]
    out = _embed_lookup(
        input_ids.reshape(-1).astype(jnp.int32),
        position_ids.reshape(-1).astype(jnp.int32),
        wte,
        wpe,
        n_tok=b * s,
        d_model=d,
    )
    return out.reshape(b, s, d)
